# baseline (speedup 1.0000x reference)
import math

import jax
import jax.numpy as jnp
from jax.experimental import pallas as pl
from jax.experimental.pallas import tpu as pltpu

_C = 196
_H = 224
_W = 224
_ITERS = 50
_GRID = 14
_MS = (10.0 / math.sqrt(_H * _W / _C)) ** 2


def _slic_kernel(x_ref, gm_ref, out_ref,
                 occ_ref, best_ref, lab_ref,
                 ycs, xcs, ccr, ccg, ccb,
                 u_ref, bm_ref, dm_ref, cnt_ref, cn0_ref, cl_ref,
                 acn, ayy, axx, arr, agg, abb):
    rowi = jax.lax.broadcasted_iota(jnp.int32, (_H, _W), 0)
    coli = jax.lax.broadcasted_iota(jnp.int32, (_H, _W), 1)
    rowf = rowi.astype(jnp.float32)
    colf = coli.astype(jnp.float32)
    ms = jnp.float32(_MS)
    inf = jnp.float32(jnp.inf)

    occ_ref[:, :] = jnp.zeros((_H, _W), jnp.int32)

    def seed_body(c, _):
        i = c // _GRID
        j = c % _GRID
        yb = 8 + 16 * i
        xb = 8 + 16 * j
        y0 = jnp.maximum(yb - 10, 0)
        y1 = jnp.minimum(yb + 10, _H)
        x0 = jnp.maximum(xb - 10, 0)
        x1 = jnp.minimum(xb + 10, _W)
        rs = jnp.clip(16 * i - 8, 0, _H - 32)
        rs = pl.multiple_of(rs, 8)
        gmw = gm_ref[pl.ds(rs, 32), :]
        occw = occ_ref[pl.ds(rs, 32), :]
        lrow = jax.lax.broadcasted_iota(jnp.int32, (32, _W), 0) + rs
        lcol = jax.lax.broadcasted_iota(jnp.int32, (32, _W), 1)
        inside = (lrow >= y0) & (lrow < y1) & (lcol >= x0) & (lcol < x1)
        mv = jnp.min(jnp.where(inside, gmw, inf))
        cand = (gmw == mv) & inside & (occw == 0)
        gflat = lrow * _W + lcol
        big = jnp.int32(_H * _W + 7)
        idx = jnp.min(jnp.where(cand, gflat, big))
        found = idx < big
        occ_ref[pl.ds(rs, 32), :] = jnp.where(
            (gflat == idx) & found, 1, occw)
        ycs[c] = jnp.where(found, idx // _W, yb)
        xcs[c] = jnp.where(found, idx % _W, xb)
        return 0

    jax.lax.fori_loop(0, _C, seed_body, 0)

    colm8 = jax.lax.broadcasted_iota(jnp.int32, (8, _W), 1)
    rowm8 = jax.lax.broadcasted_iota(jnp.int32, (8, _W), 0)

    def ccinit_body(c, _):
        y = ycs[c]
        xx = xcs[c]
        ya = pl.multiple_of((y // 8) * 8, 8)
        sel = (colm8 == xx) & (rowm8 == y - ya)
        ccr[c] = jnp.sum(jnp.where(sel, x_ref[0, pl.ds(ya, 8), :], 0.0))
        ccg[c] = jnp.sum(jnp.where(sel, x_ref[1, pl.ds(ya, 8), :], 0.0))
        ccb[c] = jnp.sum(jnp.where(sel, x_ref[2, pl.ds(ya, 8), :], 0.0))
        return 0

    jax.lax.fori_loop(0, _C, ccinit_body, 0)

    xr = x_ref[0]
    xg = x_ref[1]
    xb_ = x_ref[2]

    def _sqrt_s(v):
        return jnp.max(jnp.sqrt(jnp.full((8, 128), v, jnp.float32)))

    _BH = 32
    _NB = _H // _BH
    brow = jax.lax.broadcasted_iota(jnp.int32, (_BH, _W), 0)
    bcolf = jax.lax.broadcasted_iota(jnp.int32, (_BH, _W), 1).astype(
        jnp.float32)

    _SQM = 0.625

    def _round_i32(q):
        v = jnp.round(jnp.full((8, 128), q, jnp.float32)).astype(jnp.int32)
        return jnp.max(v)

    def _seg_update(c, cnt, sy, sx, sr, sg, sb):
        nz = cnt > 0.0
        safe = jnp.where(nz, cnt, 1.0)
        ny = jnp.clip(_round_i32(sy / safe), 0, _H - 1)
        nx = jnp.clip(_round_i32(sx / safe), 0, _W - 1)
        y_o = ycs[c]
        x_o = xcs[c]
        r_o = ccr[c]
        g_o = ccg[c]
        b_o = ccb[c]
        y_n = jnp.where(nz, ny, y_o)
        x_n = jnp.where(nz, nx, x_o)
        r_n = jnp.where(nz, sr / safe, r_o)
        g_n = jnp.where(nz, sg / safe, g_o)
        b_n = jnp.where(nz, sb / safe, b_o)
        ycs[c] = y_n
        xcs[c] = x_n
        ccr[c] = r_n
        ccg[c] = g_n
        ccb[c] = b_n
        dyf = (y_n - y_o).astype(jnp.float32)
        dxf = (x_n - x_o).astype(jnp.float32)
        drf = r_n - r_o
        dgf = g_n - g_o
        dbf = b_n - b_o
        d2 = ms * (dyf * dyf + dxf * dxf) + (drf * drf + dgf * dgf
                                             + dbf * dbf)
        dm_ref[0] = jnp.maximum(dm_ref[0], d2)

    scolf = bcolf * jnp.float32(_SQM)

    def dist_pass(do_seg):
        sd = _sqrt_s(dm_ref[0])
        for b in range(_NB):
            sroot = _sqrt_s(bm_ref[b]) + sd
            u_ref[b] = sroot * sroot * jnp.float32(1.01) + jnp.float32(1.0)
            cnt_ref[b] = 0

        def build_body(c, _):
            y = ycs[c]
            for b in range(_NB):
                dmin = jnp.maximum(
                    0, jnp.maximum(_BH * b - y, y - (_BH * b + _BH - 1)))
                dm = dmin.astype(jnp.float32)

                @pl.when(ms * (dm * dm) <= u_ref[b])
                def _(b=b, c=c):
                    k = cnt_ref[b]
                    cl_ref[b, k] = c
                    cnt_ref[b] = k + 1

            if do_seg:
                acn[c] = jnp.float32(0.0)
                ayy[c] = jnp.float32(0.0)
                axx[c] = jnp.float32(0.0)
                arr[c] = jnp.float32(0.0)
                agg[c] = jnp.float32(0.0)
                abb[c] = jnp.float32(0.0)
            return 0

        jax.lax.fori_loop(0, _C, build_body, 0)

        for b in range(_NB):
            k = cnt_ref[b]
            cn0_ref[b] = k
            pad = (-k) % 4
            for t in range(3):
                @pl.when(t < pad)
                def _(b=b, k=k, t=t):
                    cl_ref[b, k + t] = cl_ref[b, 0]

            cnt_ref[b] = k + pad

        for b in range(_NB):
            r0 = _BH * b
            rowfb = (brow + r0).astype(jnp.float32)
            srowfb = rowfb * jnp.float32(_SQM)
            xr_b = x_ref[0, pl.ds(r0, _BH), :]
            xg_b = x_ref[1, pl.ds(r0, _BH), :]
            xb_b = x_ref[2, pl.ds(r0, _BH), :]

            def c_body(g, carry, srowfb=srowfb, b=b,
                       xr_b=xr_b, xg_b=xg_b, xb_b=xb_b):
                bb, ll = carry
                k = g * 4

                def one(c):
                    sy = srowfb - ycs[c].astype(jnp.float32) * jnp.float32(
                        _SQM)
                    sx = scolf - xcs[c].astype(jnp.float32) * jnp.float32(
                        _SQM)
                    d0 = xr_b - ccr[c]
                    d1 = xg_b - ccg[c]
                    d2 = xb_b - ccb[c]
                    return ((d0 * d0 + d1 * d1) + d2 * d2) + (
                        sy * sy + sx * sx)

                c0 = cl_ref[b, k]
                c1 = cl_ref[b, k + 1]
                c2 = cl_ref[b, k + 2]
                c3 = cl_ref[b, k + 3]
                v0 = one(c0)
                v1 = one(c1)
                v2 = one(c2)
                v3 = one(c3)
                t01 = v1 < v0
                va = jnp.where(t01, v1, v0)
                la = jnp.where(t01, c1, c0)
                t23 = v3 < v2
                vb = jnp.where(t23, v3, v2)
                lb = jnp.where(t23, c3, c2)
                tab = vb < va
                vg = jnp.where(tab, vb, va)
                lg = jnp.where(tab, lb, la)
                upd = vg < bb
                return jnp.where(upd, vg, bb), jnp.where(upd, lg, ll)

            bb, ll = jax.lax.fori_loop(
                0, cnt_ref[b] // 4, c_body,
                (jnp.full((_BH, _W), inf, jnp.float32),
                 jnp.zeros((_BH, _W), jnp.int32)))
            best_ref[pl.ds(r0, _BH), :] = bb
            lab_ref[pl.ds(r0, _BH), :] = ll
            bm_ref[b] = jnp.max(bb)

            if do_seg:
                def s_body(k2, _, b=b, ll=ll, rowfb=rowfb,
                           xr_b=xr_b, xg_b=xg_b, xb_b=xb_b):
                    c = cl_ref[b, k2]
                    m = ll == c
                    acn[c] = acn[c] + jnp.sum(jnp.where(m, 1.0, 0.0))
                    ayy[c] = ayy[c] + jnp.sum(jnp.where(m, rowfb, 0.0))
                    axx[c] = axx[c] + jnp.sum(jnp.where(m, bcolf, 0.0))
                    arr[c] = arr[c] + jnp.sum(jnp.where(m, xr_b, 0.0))
                    agg[c] = agg[c] + jnp.sum(jnp.where(m, xg_b, 0.0))
                    abb[c] = abb[c] + jnp.sum(jnp.where(m, xb_b, 0.0))
                    return 0

                jax.lax.fori_loop(0, cn0_ref[b], s_body, 0)

    def update_pass():
        dm_ref[0] = jnp.float32(0.0)

        def upd_body(c, _):
            _seg_update(c, acn[c], ayy[c], axx[c], arr[c], agg[c], abb[c])
            return 0

        jax.lax.fori_loop(0, _C, upd_body, 0)

    dm_ref[0] = jnp.float32(0.0)
    for b in range(_NB):
        bm_ref[b] = jnp.float32(260.0)

    def it_body(t, _):
        dist_pass(True)
        update_pass()
        return 0

    jax.lax.fori_loop(0, _ITERS - 1, it_body, 0)
    dist_pass(False)
    out_ref[0] = lab_ref[:, :]


def kernel(x, grad_map):
    if grad_map.ndim == 3:
        grad_map = grad_map[:, None]
    f = pl.pallas_call(
        _slic_kernel,
        out_shape=jax.ShapeDtypeStruct((1, _H, _W), jnp.int32),
        scratch_shapes=[
            pltpu.VMEM((_H, _W), jnp.int32),
            pltpu.VMEM((_H, _W), jnp.float32),
            pltpu.VMEM((_H, _W), jnp.int32),
            pltpu.SMEM((_C,), jnp.int32),
            pltpu.SMEM((_C,), jnp.int32),
            pltpu.SMEM((_C,), jnp.float32),
            pltpu.SMEM((_C,), jnp.float32),
            pltpu.SMEM((_C,), jnp.float32),
            pltpu.SMEM((8,), jnp.float32),
            pltpu.SMEM((8,), jnp.float32),
            pltpu.SMEM((1,), jnp.float32),
            pltpu.SMEM((8,), jnp.int32),
            pltpu.SMEM((8,), jnp.int32),
            pltpu.SMEM((_H // 32, _C), jnp.int32),
            pltpu.SMEM((_C,), jnp.float32),
            pltpu.SMEM((_C,), jnp.float32),
            pltpu.SMEM((_C,), jnp.float32),
            pltpu.SMEM((_C,), jnp.float32),
            pltpu.SMEM((_C,), jnp.float32),
            pltpu.SMEM((_C,), jnp.float32),
        ],
    )
    return f(x[0], grad_map[0, 0])


# speedup vs baseline: 2.2140x; 2.2140x over previous
import math

import jax
import jax.numpy as jnp
from jax.experimental import pallas as pl
from jax.experimental.pallas import tpu as pltpu

_C = 196
_H = 224
_W = 224
_ITERS = 50
_GRID = 14
_MS = (10.0 / math.sqrt(_H * _W / _C)) ** 2


def _slic_kernel(x_ref, gm_ref, out_ref,
                 occ_ref, best_ref, lab_ref,
                 ycs, xcs, ccr, ccg, ccb,
                 u_ref, dm_ref, cnt_ref, cl_ref):
    rowi = jax.lax.broadcasted_iota(jnp.int32, (_H, _W), 0)
    coli = jax.lax.broadcasted_iota(jnp.int32, (_H, _W), 1)
    rowf = rowi.astype(jnp.float32)
    colf = coli.astype(jnp.float32)
    ms = jnp.float32(_MS)
    inf = jnp.float32(jnp.inf)

    occ_ref[:, :] = jnp.zeros((_H, _W), jnp.int32)

    def seed_body(c, _):
        i = c // _GRID
        j = c % _GRID
        yb = 8 + 16 * i
        xb = 8 + 16 * j
        y0 = jnp.maximum(yb - 10, 0)
        y1 = jnp.minimum(yb + 10, _H)
        x0 = jnp.maximum(xb - 10, 0)
        x1 = jnp.minimum(xb + 10, _W)
        rs = jnp.clip(16 * i - 8, 0, _H - 32)
        rs = pl.multiple_of(rs, 8)
        gmw = gm_ref[pl.ds(rs, 32), :]
        occw = occ_ref[pl.ds(rs, 32), :]
        lrow = jax.lax.broadcasted_iota(jnp.int32, (32, _W), 0) + rs
        lcol = jax.lax.broadcasted_iota(jnp.int32, (32, _W), 1)
        inside = (lrow >= y0) & (lrow < y1) & (lcol >= x0) & (lcol < x1)
        mv = jnp.min(jnp.where(inside, gmw, inf))
        cand = (gmw == mv) & inside & (occw == 0)
        gflat = lrow * _W + lcol
        big = jnp.int32(_H * _W + 7)
        idx = jnp.min(jnp.where(cand, gflat, big))
        found = idx < big
        occ_ref[pl.ds(rs, 32), :] = jnp.where(
            (gflat == idx) & found, 1, occw)
        ycs[c] = jnp.where(found, idx // _W, yb)
        xcs[c] = jnp.where(found, idx % _W, xb)
        return 0

    jax.lax.fori_loop(0, _C, seed_body, 0)

    colm8 = jax.lax.broadcasted_iota(jnp.int32, (8, _W), 1)
    rowm8 = jax.lax.broadcasted_iota(jnp.int32, (8, _W), 0)

    def ccinit_body(c, _):
        y = ycs[c]
        xx = xcs[c]
        ya = pl.multiple_of((y // 8) * 8, 8)
        sel = (colm8 == xx) & (rowm8 == y - ya)
        ccr[c] = jnp.sum(jnp.where(sel, x_ref[0, pl.ds(ya, 8), :], 0.0))
        ccg[c] = jnp.sum(jnp.where(sel, x_ref[1, pl.ds(ya, 8), :], 0.0))
        ccb[c] = jnp.sum(jnp.where(sel, x_ref[2, pl.ds(ya, 8), :], 0.0))
        return 0

    jax.lax.fori_loop(0, _C, ccinit_body, 0)

    xr = x_ref[0]
    xg = x_ref[1]
    xb_ = x_ref[2]

    def _sqrt_s(v):
        return jnp.max(jnp.sqrt(jnp.full((8, 128), v, jnp.float32)))

    _BH = 32
    _NB = _H // _BH
    brow = jax.lax.broadcasted_iota(jnp.int32, (_BH, _W), 0)
    bcolf = jax.lax.broadcasted_iota(jnp.int32, (_BH, _W), 1).astype(
        jnp.float32)

    _SQM = 0.625

    def dist_pass():
        u = u_ref[0]

        for b in range(_NB):
            cnt_ref[b] = 0

        def build_body(c, _):
            y = ycs[c]
            for b in range(_NB):
                dmin = jnp.maximum(
                    0, jnp.maximum(_BH * b - y, y - (_BH * b + _BH - 1)))
                dm = dmin.astype(jnp.float32)

                @pl.when(ms * (dm * dm) <= u)
                def _():
                    k = cnt_ref[b]
                    cl_ref[b, k] = c
                    cnt_ref[b] = k + 1

            return 0

        jax.lax.fori_loop(0, _C, build_body, 0)

        for b in range(_NB):
            k = cnt_ref[b]
            pad = (-k) % 4
            for t in range(3):
                @pl.when(t < pad)
                def _(b=b, k=k, t=t):
                    cl_ref[b, k + t] = cl_ref[b, 0]

            cnt_ref[b] = k + pad

        scolf = bcolf * jnp.float32(_SQM)
        for b in range(_NB):
            r0 = _BH * b
            srowfb = (brow + r0).astype(jnp.float32) * jnp.float32(_SQM)
            xr_b = x_ref[0, pl.ds(r0, _BH), :]
            xg_b = x_ref[1, pl.ds(r0, _BH), :]
            xb_b = x_ref[2, pl.ds(r0, _BH), :]

            def c_body(g, carry, srowfb=srowfb, b=b,
                       xr_b=xr_b, xg_b=xg_b, xb_b=xb_b):
                bb, ll = carry
                k = g * 4

                def one(c):
                    sy = srowfb - ycs[c].astype(jnp.float32) * jnp.float32(
                        _SQM)
                    sx = scolf - xcs[c].astype(jnp.float32) * jnp.float32(
                        _SQM)
                    d0 = xr_b - ccr[c]
                    d1 = xg_b - ccg[c]
                    d2 = xb_b - ccb[c]
                    return ((d0 * d0 + d1 * d1) + d2 * d2) + (
                        sy * sy + sx * sx)

                c0 = cl_ref[b, k]
                c1 = cl_ref[b, k + 1]
                c2 = cl_ref[b, k + 2]
                c3 = cl_ref[b, k + 3]
                v0 = one(c0)
                v1 = one(c1)
                v2 = one(c2)
                v3 = one(c3)
                t01 = v1 < v0
                va = jnp.where(t01, v1, v0)
                la = jnp.where(t01, c1, c0)
                t23 = v3 < v2
                vb = jnp.where(t23, v3, v2)
                lb = jnp.where(t23, c3, c2)
                tab = vb < va
                vg = jnp.where(tab, vb, va)
                lg = jnp.where(tab, lb, la)
                upd = vg < bb
                return jnp.where(upd, vg, bb), jnp.where(upd, lg, ll)

            bb, ll = jax.lax.fori_loop(
                0, cnt_ref[b] // 4, c_body,
                (jnp.full((_BH, _W), inf, jnp.float32),
                 jnp.zeros((_BH, _W), jnp.int32)))
            best_ref[pl.ds(r0, _BH), :] = bb
            lab_ref[pl.ds(r0, _BH), :] = ll

    def _round_i32(q):
        v = jnp.round(jnp.full((8, 128), q, jnp.float32)).astype(jnp.int32)
        return jnp.max(v)

    def _seg_update(c, cnt, sy, sx, sr, sg, sb):
        nz = cnt > 0.0
        safe = jnp.where(nz, cnt, 1.0)
        ny = jnp.clip(_round_i32(sy / safe), 0, _H - 1)
        nx = jnp.clip(_round_i32(sx / safe), 0, _W - 1)
        y_o = ycs[c]
        x_o = xcs[c]
        r_o = ccr[c]
        g_o = ccg[c]
        b_o = ccb[c]
        y_n = jnp.where(nz, ny, y_o)
        x_n = jnp.where(nz, nx, x_o)
        r_n = jnp.where(nz, sr / safe, r_o)
        g_n = jnp.where(nz, sg / safe, g_o)
        b_n = jnp.where(nz, sb / safe, b_o)
        ycs[c] = y_n
        xcs[c] = x_n
        ccr[c] = r_n
        ccg[c] = g_n
        ccb[c] = b_n
        dyf = (y_n - y_o).astype(jnp.float32)
        dxf = (x_n - x_o).astype(jnp.float32)
        drf = r_n - r_o
        dgf = g_n - g_o
        dbf = b_n - b_o
        d2 = ms * (dyf * dyf + dxf * dxf) + (drf * drf + dgf * dgf
                                             + dbf * dbf)
        dm_ref[0] = jnp.maximum(dm_ref[0], d2)

    _WROW = {
        w: jax.lax.broadcasted_iota(jnp.int32, (w, _W), 0) for w in (40, 64)
    }
    _WCOLF = {
        w: jax.lax.broadcasted_iota(jnp.int32, (w, _W), 1).astype(
            jnp.float32) for w in (40, 64)
    }

    def seg_pass():
        bmax = jnp.max(best_ref[:, :])
        small40 = bmax <= jnp.float32(_MS * 16 * 16)
        small64 = bmax <= jnp.float32(_MS * 28 * 28)
        dm_ref[0] = jnp.float32(0.0)

        def _win_sums(c, win, rad):
            y = ycs[c]
            st = jnp.clip(((y - rad) // 8) * 8, 0, _H - win)
            st = pl.multiple_of(st, 8)
            m = lab_ref[pl.ds(st, win), :] == c
            rowfw = (_WROW[win] + st).astype(jnp.float32)
            cnt = jnp.sum(jnp.where(m, 1.0, 0.0))
            sy = jnp.sum(jnp.where(m, rowfw, 0.0))
            sx = jnp.sum(jnp.where(m, _WCOLF[win], 0.0))
            sr = jnp.sum(jnp.where(m, x_ref[0, pl.ds(st, win), :], 0.0))
            sg = jnp.sum(jnp.where(m, x_ref[1, pl.ds(st, win), :], 0.0))
            sb = jnp.sum(jnp.where(m, x_ref[2, pl.ds(st, win), :], 0.0))
            return cnt, sy, sx, sr, sg, sb

        def c_body_w40(k, _):
            base = k * 4
            sums = [_win_sums(base + t, 40, 16) for t in range(4)]
            for t in range(4):
                _seg_update(base + t, *sums[t])
            return 0

        def c_body_win(k, _):
            base = k * 4
            sums = [_win_sums(base + t, 64, 28) for t in range(4)]
            for t in range(4):
                _seg_update(base + t, *sums[t])
            return 0

        def c_body_full(c, _):
            m = lab_ref[:, :] == c
            cnt = jnp.sum(jnp.where(m, 1.0, 0.0))
            sy = jnp.sum(jnp.where(m, rowf, 0.0))
            sx = jnp.sum(jnp.where(m, colf, 0.0))
            sr = jnp.sum(jnp.where(m, xr, 0.0))
            sg = jnp.sum(jnp.where(m, xg, 0.0))
            sb = jnp.sum(jnp.where(m, xb_, 0.0))
            _seg_update(c, cnt, sy, sx, sr, sg, sb)
            return 0

        jax.lax.cond(
            small40,
            lambda: jax.lax.fori_loop(0, _C // 4, c_body_w40, 0),
            lambda: jax.lax.cond(
                small64,
                lambda: jax.lax.fori_loop(0, _C // 4, c_body_win, 0),
                lambda: jax.lax.fori_loop(0, _C, c_body_full, 0),
            ),
        )
        sroot = _sqrt_s(bmax) + _sqrt_s(dm_ref[0])
        u_ref[0] = sroot * sroot * jnp.float32(1.01) + jnp.float32(1.0)

    u_ref[0] = jnp.float32(260.0)

    def it_body(t, _):
        dist_pass()
        seg_pass()
        return 0

    jax.lax.fori_loop(0, _ITERS - 1, it_body, 0)
    dist_pass()
    out_ref[0] = lab_ref[:, :]


def kernel(x, grad_map):
    if grad_map.ndim == 3:
        grad_map = grad_map[:, None]
    f = pl.pallas_call(
        _slic_kernel,
        out_shape=jax.ShapeDtypeStruct((1, _H, _W), jnp.int32),
        scratch_shapes=[
            pltpu.VMEM((_H, _W), jnp.int32),
            pltpu.VMEM((_H, _W), jnp.float32),
            pltpu.VMEM((_H, _W), jnp.int32),
            pltpu.SMEM((_C,), jnp.int32),
            pltpu.SMEM((_C,), jnp.int32),
            pltpu.SMEM((_C,), jnp.float32),
            pltpu.SMEM((_C,), jnp.float32),
            pltpu.SMEM((_C,), jnp.float32),
            pltpu.SMEM((1,), jnp.float32),
            pltpu.SMEM((1,), jnp.float32),
            pltpu.SMEM((8,), jnp.int32),
            pltpu.SMEM((_H // 32, _C), jnp.int32),
        ],
    )
    return f(x[0], grad_map[0, 0])


# speedup vs baseline: 5.9506x; 2.6877x over previous
import math

import jax
import jax.numpy as jnp
from jax.experimental import pallas as pl
from jax.experimental.pallas import tpu as pltpu

_C = 196
_H = 224
_W = 224
_ITERS = 50
_GRID = 14
_MS = (10.0 / math.sqrt(_H * _W / _C)) ** 2


def _slic_kernel(x_ref, gm_ref, out_ref,
                 occ_ref, best_ref, lab_ref,
                 ycs, xcs, ccr, ccg, ccb,
                 u_ref, dm_ref, cnt_ref, cl_ref):
    rowi = jax.lax.broadcasted_iota(jnp.int32, (_H, _W), 0)
    coli = jax.lax.broadcasted_iota(jnp.int32, (_H, _W), 1)
    rowf = rowi.astype(jnp.float32)
    colf = coli.astype(jnp.float32)
    ms = jnp.float32(_MS)
    inf = jnp.float32(jnp.inf)

    occ_ref[:, :] = jnp.zeros((_H, _W), jnp.int32)

    def seed_body(c, _):
        i = c // _GRID
        j = c % _GRID
        yb = 8 + 16 * i
        xb = 8 + 16 * j
        y0 = jnp.maximum(yb - 10, 0)
        y1 = jnp.minimum(yb + 10, _H)
        x0 = jnp.maximum(xb - 10, 0)
        x1 = jnp.minimum(xb + 10, _W)
        rs = jnp.clip(16 * i - 8, 0, _H - 32)
        rs = pl.multiple_of(rs, 8)
        gmw = gm_ref[pl.ds(rs, 32), :]
        occw = occ_ref[pl.ds(rs, 32), :]
        lrow = jax.lax.broadcasted_iota(jnp.int32, (32, _W), 0) + rs
        lcol = jax.lax.broadcasted_iota(jnp.int32, (32, _W), 1)
        inside = (lrow >= y0) & (lrow < y1) & (lcol >= x0) & (lcol < x1)
        mv = jnp.min(jnp.where(inside, gmw, inf))
        cand = (gmw == mv) & inside & (occw == 0)
        gflat = lrow * _W + lcol
        big = jnp.int32(_H * _W + 7)
        idx = jnp.min(jnp.where(cand, gflat, big))
        found = idx < big
        occ_ref[pl.ds(rs, 32), :] = jnp.where(
            (gflat == idx) & found, 1, occw)
        ycs[c] = jnp.where(found, idx // _W, yb)
        xcs[c] = jnp.where(found, idx % _W, xb)
        return 0

    jax.lax.fori_loop(0, _C, seed_body, 0)

    colm8 = jax.lax.broadcasted_iota(jnp.int32, (8, _W), 1)
    rowm8 = jax.lax.broadcasted_iota(jnp.int32, (8, _W), 0)

    def ccinit_body(c, _):
        y = ycs[c]
        xx = xcs[c]
        ya = pl.multiple_of((y // 8) * 8, 8)
        sel = (colm8 == xx) & (rowm8 == y - ya)
        ccr[c] = jnp.sum(jnp.where(sel, x_ref[0, pl.ds(ya, 8), :], 0.0))
        ccg[c] = jnp.sum(jnp.where(sel, x_ref[1, pl.ds(ya, 8), :], 0.0))
        ccb[c] = jnp.sum(jnp.where(sel, x_ref[2, pl.ds(ya, 8), :], 0.0))
        return 0

    jax.lax.fori_loop(0, _C, ccinit_body, 0)

    xr = x_ref[0]
    xg = x_ref[1]
    xb_ = x_ref[2]

    def _sqrt_s(v):
        return jnp.max(jnp.sqrt(jnp.full((8, 128), v, jnp.float32)))

    _BH = 32
    _NB = _H // _BH
    brow = jax.lax.broadcasted_iota(jnp.int32, (_BH, _W), 0)
    bcolf = jax.lax.broadcasted_iota(jnp.int32, (_BH, _W), 1).astype(
        jnp.float32)

    _SQM = 0.625

    def dist_pass():
        u = u_ref[0]

        for b in range(_NB):
            cnt_ref[b] = 0

        def build_body(c, _):
            y = ycs[c]
            for b in range(_NB):
                dmin = jnp.maximum(
                    0, jnp.maximum(_BH * b - y, y - (_BH * b + _BH - 1)))
                dm = dmin.astype(jnp.float32)

                @pl.when(ms * (dm * dm) <= u)
                def _():
                    k = cnt_ref[b]
                    cl_ref[b, k] = c
                    cnt_ref[b] = k + 1

            return 0

        jax.lax.fori_loop(0, _C, build_body, 0)

        for b in range(_NB):
            k = cnt_ref[b]
            pad = (-k) % 4
            for t in range(3):
                @pl.when(t < pad)
                def _(b=b, k=k, t=t):
                    cl_ref[b, k + t] = cl_ref[b, 0]

            cnt_ref[b] = k + pad

        scolf = bcolf * jnp.float32(_SQM)
        for b in range(_NB):
            r0 = _BH * b
            srowfb = (brow + r0).astype(jnp.float32) * jnp.float32(_SQM)
            xr_b = x_ref[0, pl.ds(r0, _BH), :]
            xg_b = x_ref[1, pl.ds(r0, _BH), :]
            xb_b = x_ref[2, pl.ds(r0, _BH), :]

            def c_body(g, carry, srowfb=srowfb, b=b,
                       xr_b=xr_b, xg_b=xg_b, xb_b=xb_b):
                bb, ll = carry
                k = g * 4

                def one(c):
                    sy = srowfb - ycs[c].astype(jnp.float32) * jnp.float32(
                        _SQM)
                    sx = scolf - xcs[c].astype(jnp.float32) * jnp.float32(
                        _SQM)
                    d0 = xr_b - ccr[c]
                    d1 = xg_b - ccg[c]
                    d2 = xb_b - ccb[c]
                    return ((d0 * d0 + d1 * d1) + d2 * d2) + (
                        sy * sy + sx * sx)

                c0 = cl_ref[b, k]
                c1 = cl_ref[b, k + 1]
                c2 = cl_ref[b, k + 2]
                c3 = cl_ref[b, k + 3]
                v0 = one(c0)
                v1 = one(c1)
                v2 = one(c2)
                v3 = one(c3)
                t01 = v1 < v0
                va = jnp.where(t01, v1, v0)
                la = jnp.where(t01, c1, c0)
                t23 = v3 < v2
                vb = jnp.where(t23, v3, v2)
                lb = jnp.where(t23, c3, c2)
                tab = vb < va
                vg = jnp.where(tab, vb, va)
                lg = jnp.where(tab, lb, la)
                upd = vg < bb
                return jnp.where(upd, vg, bb), jnp.where(upd, lg, ll)

            bb, ll = jax.lax.fori_loop(
                0, cnt_ref[b] // 4, c_body,
                (jnp.full((_BH, _W), inf, jnp.float32),
                 jnp.zeros((_BH, _W), jnp.int32)))
            best_ref[pl.ds(r0, _BH), :] = bb
            lab_ref[pl.ds(r0, _BH), :] = ll

    def _round_i32(q):
        v = jnp.round(jnp.full((8, 128), q, jnp.float32)).astype(jnp.int32)
        return jnp.max(v)

    def _seg_update(c, cnt, sy, sx, sr, sg, sb):
        nz = cnt > 0.0
        safe = jnp.where(nz, cnt, 1.0)
        ny = jnp.clip(_round_i32(sy / safe), 0, _H - 1)
        nx = jnp.clip(_round_i32(sx / safe), 0, _W - 1)
        y_o = ycs[c]
        x_o = xcs[c]
        r_o = ccr[c]
        g_o = ccg[c]
        b_o = ccb[c]
        y_n = jnp.where(nz, ny, y_o)
        x_n = jnp.where(nz, nx, x_o)
        r_n = jnp.where(nz, sr / safe, r_o)
        g_n = jnp.where(nz, sg / safe, g_o)
        b_n = jnp.where(nz, sb / safe, b_o)
        ycs[c] = y_n
        xcs[c] = x_n
        ccr[c] = r_n
        ccg[c] = g_n
        ccb[c] = b_n
        dyf = (y_n - y_o).astype(jnp.float32)
        dxf = (x_n - x_o).astype(jnp.float32)
        drf = r_n - r_o
        dgf = g_n - g_o
        dbf = b_n - b_o
        d2 = ms * (dyf * dyf + dxf * dxf) + (drf * drf + dgf * dgf
                                             + dbf * dbf)
        dm_ref[0] = jnp.maximum(dm_ref[0], d2)

    _WROW = {
        w: jax.lax.broadcasted_iota(jnp.int32, (w, _W), 0) for w in (40, 64)
    }
    _WCOLF = {
        w: jax.lax.broadcasted_iota(jnp.int32, (w, _W), 1).astype(
            jnp.float32) for w in (40, 64)
    }

    def seg_pass():
        bmax = jnp.max(best_ref[:, :])
        small40 = bmax <= jnp.float32(_MS * 16 * 16)
        small64 = bmax <= jnp.float32(_MS * 28 * 28)
        dm_ref[0] = jnp.float32(0.0)

        def _win_sums(c, win, rad):
            y = ycs[c]
            st = jnp.clip(((y - rad) // 8) * 8, 0, _H - win)
            st = pl.multiple_of(st, 8)
            m = lab_ref[pl.ds(st, win), :] == c
            rowfw = (_WROW[win] + st).astype(jnp.float32)
            cnt = jnp.sum(jnp.where(m, 1.0, 0.0))
            sy = jnp.sum(jnp.where(m, rowfw, 0.0))
            sx = jnp.sum(jnp.where(m, _WCOLF[win], 0.0))
            sr = jnp.sum(jnp.where(m, x_ref[0, pl.ds(st, win), :], 0.0))
            sg = jnp.sum(jnp.where(m, x_ref[1, pl.ds(st, win), :], 0.0))
            sb = jnp.sum(jnp.where(m, x_ref[2, pl.ds(st, win), :], 0.0))
            return cnt, sy, sx, sr, sg, sb

        def c_body_w40(k, _):
            base = k * 4
            sums = [_win_sums(base + t, 40, 16) for t in range(4)]
            for t in range(4):
                _seg_update(base + t, *sums[t])
            return 0

        def c_body_win(k, _):
            base = k * 4
            sums = [_win_sums(base + t, 64, 28) for t in range(4)]
            for t in range(4):
                _seg_update(base + t, *sums[t])
            return 0

        def c_body_full(c, _):
            m = lab_ref[:, :] == c
            cnt = jnp.sum(jnp.where(m, 1.0, 0.0))
            sy = jnp.sum(jnp.where(m, rowf, 0.0))
            sx = jnp.sum(jnp.where(m, colf, 0.0))
            sr = jnp.sum(jnp.where(m, xr, 0.0))
            sg = jnp.sum(jnp.where(m, xg, 0.0))
            sb = jnp.sum(jnp.where(m, xb_, 0.0))
            _seg_update(c, cnt, sy, sx, sr, sg, sb)
            return 0

        jax.lax.cond(
            small40,
            lambda: jax.lax.fori_loop(0, _C // 4, c_body_w40, 0),
            lambda: jax.lax.cond(
                small64,
                lambda: jax.lax.fori_loop(0, _C // 4, c_body_win, 0),
                lambda: jax.lax.fori_loop(0, _C, c_body_full, 0),
            ),
        )
        sroot = _sqrt_s(bmax) + _sqrt_s(dm_ref[0])
        u_ref[0] = sroot * sroot * jnp.float32(1.01) + jnp.float32(1.0)

    u_ref[0] = jnp.float32(260.0)
    dm_ref[0] = jnp.float32(1.0)

    def it_body(t, _):
        jax.lax.cond(
            dm_ref[0] != 0.0,
            lambda: (dist_pass(), seg_pass(), 0)[2],
            lambda: 0,
        )
        return 0

    jax.lax.fori_loop(0, _ITERS - 1, it_body, 0)
    dist_pass()
    out_ref[0] = lab_ref[:, :]


def kernel(x, grad_map):
    if grad_map.ndim == 3:
        grad_map = grad_map[:, None]
    f = pl.pallas_call(
        _slic_kernel,
        out_shape=jax.ShapeDtypeStruct((1, _H, _W), jnp.int32),
        scratch_shapes=[
            pltpu.VMEM((_H, _W), jnp.int32),
            pltpu.VMEM((_H, _W), jnp.float32),
            pltpu.VMEM((_H, _W), jnp.int32),
            pltpu.SMEM((_C,), jnp.int32),
            pltpu.SMEM((_C,), jnp.int32),
            pltpu.SMEM((_C,), jnp.float32),
            pltpu.SMEM((_C,), jnp.float32),
            pltpu.SMEM((_C,), jnp.float32),
            pltpu.SMEM((1,), jnp.float32),
            pltpu.SMEM((1,), jnp.float32),
            pltpu.SMEM((8,), jnp.int32),
            pltpu.SMEM((_H // 32, _C), jnp.int32),
        ],
    )
    return f(x[0], grad_map[0, 0])


# speedup vs baseline: 6.7543x; 1.1351x over previous
import math

import jax
import jax.numpy as jnp
from jax.experimental import pallas as pl
from jax.experimental.pallas import tpu as pltpu

_C = 196
_H = 224
_W = 224
_ITERS = 50
_GRID = 14
_MS = (10.0 / math.sqrt(_H * _W / _C)) ** 2


def _slic_kernel(x_ref, gm_ref, out_ref,
                 occ_ref, best_ref, lab_ref,
                 ycs, xcs, ccr, ccg, ccb,
                 u_ref, dm_ref, cnt_ref, cl_ref):
    rowi = jax.lax.broadcasted_iota(jnp.int32, (_H, _W), 0)
    coli = jax.lax.broadcasted_iota(jnp.int32, (_H, _W), 1)
    rowf = rowi.astype(jnp.float32)
    colf = coli.astype(jnp.float32)
    ms = jnp.float32(_MS)
    inf = jnp.float32(jnp.inf)

    occ_ref[:, :] = jnp.zeros((_H, _W), jnp.int32)

    def seed_body(c, _):
        i = c // _GRID
        j = c % _GRID
        yb = 8 + 16 * i
        xb = 8 + 16 * j
        y0 = jnp.maximum(yb - 10, 0)
        y1 = jnp.minimum(yb + 10, _H)
        x0 = jnp.maximum(xb - 10, 0)
        x1 = jnp.minimum(xb + 10, _W)
        rs = jnp.clip(16 * i - 8, 0, _H - 32)
        rs = pl.multiple_of(rs, 8)
        gmw = gm_ref[pl.ds(rs, 32), :]
        occw = occ_ref[pl.ds(rs, 32), :]
        lrow = jax.lax.broadcasted_iota(jnp.int32, (32, _W), 0) + rs
        lcol = jax.lax.broadcasted_iota(jnp.int32, (32, _W), 1)
        inside = (lrow >= y0) & (lrow < y1) & (lcol >= x0) & (lcol < x1)
        mv = jnp.min(jnp.where(inside, gmw, inf))
        cand = (gmw == mv) & inside & (occw == 0)
        gflat = lrow * _W + lcol
        big = jnp.int32(_H * _W + 7)
        idx = jnp.min(jnp.where(cand, gflat, big))
        found = idx < big
        occ_ref[pl.ds(rs, 32), :] = jnp.where(
            (gflat == idx) & found, 1, occw)
        ycs[c] = jnp.where(found, idx // _W, yb)
        xcs[c] = jnp.where(found, idx % _W, xb)
        return 0

    jax.lax.fori_loop(0, _C, seed_body, 0)

    colm8 = jax.lax.broadcasted_iota(jnp.int32, (8, _W), 1)
    rowm8 = jax.lax.broadcasted_iota(jnp.int32, (8, _W), 0)

    def ccinit_body(c, _):
        y = ycs[c]
        xx = xcs[c]
        ya = pl.multiple_of((y // 8) * 8, 8)
        sel = (colm8 == xx) & (rowm8 == y - ya)
        ccr[c] = jnp.sum(jnp.where(sel, x_ref[0, pl.ds(ya, 8), :], 0.0))
        ccg[c] = jnp.sum(jnp.where(sel, x_ref[1, pl.ds(ya, 8), :], 0.0))
        ccb[c] = jnp.sum(jnp.where(sel, x_ref[2, pl.ds(ya, 8), :], 0.0))
        return 0

    jax.lax.fori_loop(0, _C, ccinit_body, 0)

    xr = x_ref[0]
    xg = x_ref[1]
    xb_ = x_ref[2]

    def _sqrt_s(v):
        return jnp.max(jnp.sqrt(jnp.full((8, 128), v, jnp.float32)))

    _BH = 32
    _NB = _H // _BH
    brow = jax.lax.broadcasted_iota(jnp.int32, (_BH, _W), 0)
    bcolf = jax.lax.broadcasted_iota(jnp.int32, (_BH, _W), 1).astype(
        jnp.float32)

    _SQM = 0.625

    def dist_pass():
        u = u_ref[0]

        for b in range(_NB):
            cnt_ref[b] = 0

        def build_body(c, _):
            y = ycs[c]
            for b in range(_NB):
                dmin = jnp.maximum(
                    0, jnp.maximum(_BH * b - y, y - (_BH * b + _BH - 1)))
                dm = dmin.astype(jnp.float32)

                @pl.when(ms * (dm * dm) <= u)
                def _():
                    k = cnt_ref[b]
                    cl_ref[b, k] = c
                    cnt_ref[b] = k + 1

            return 0

        jax.lax.fori_loop(0, _C, build_body, 0)

        for b in range(_NB):
            k = cnt_ref[b]
            pad = (-k) % 8
            for t in range(7):
                @pl.when(t < pad)
                def _(b=b, k=k, t=t):
                    cl_ref[b, k + t] = cl_ref[b, 0]

            cnt_ref[b] = k + pad

        scolf = bcolf * jnp.float32(_SQM)
        for b in range(_NB):
            r0 = _BH * b
            srowfb = (brow + r0).astype(jnp.float32) * jnp.float32(_SQM)
            xr_b = x_ref[0, pl.ds(r0, _BH), :]
            xg_b = x_ref[1, pl.ds(r0, _BH), :]
            xb_b = x_ref[2, pl.ds(r0, _BH), :]

            def c_body(g, carry, srowfb=srowfb, b=b,
                       xr_b=xr_b, xg_b=xg_b, xb_b=xb_b):
                bb, ll = carry
                k = g * 8

                def one(c):
                    sy = srowfb - ycs[c].astype(jnp.float32) * jnp.float32(
                        _SQM)
                    sx = scolf - xcs[c].astype(jnp.float32) * jnp.float32(
                        _SQM)
                    d0 = xr_b - ccr[c]
                    d1 = xg_b - ccg[c]
                    d2 = xb_b - ccb[c]
                    return ((d0 * d0 + d1 * d1) + d2 * d2) + (
                        sy * sy + sx * sx)

                cs = [cl_ref[b, k + t] for t in range(8)]
                vs = [one(c) for c in cs]
                while len(vs) > 1:
                    nvs, ncs = [], []
                    for i in range(0, len(vs), 2):
                        t = vs[i + 1] < vs[i]
                        nvs.append(jnp.where(t, vs[i + 1], vs[i]))
                        ncs.append(jnp.where(t, cs[i + 1], cs[i]))
                    vs, cs = nvs, ncs
                upd = vs[0] < bb
                return jnp.where(upd, vs[0], bb), jnp.where(upd, cs[0], ll)

            bb, ll = jax.lax.fori_loop(
                0, cnt_ref[b] // 8, c_body,
                (jnp.full((_BH, _W), inf, jnp.float32),
                 jnp.zeros((_BH, _W), jnp.int32)))
            best_ref[pl.ds(r0, _BH), :] = bb
            lab_ref[pl.ds(r0, _BH), :] = ll

    def _round_i32(q):
        v = jnp.round(jnp.full((8, 128), q, jnp.float32)).astype(jnp.int32)
        return jnp.max(v)

    def _seg_update(c, cnt, sy, sx, sr, sg, sb):
        nz = cnt > 0.0
        safe = jnp.where(nz, cnt, 1.0)
        ny = jnp.clip(_round_i32(sy / safe), 0, _H - 1)
        nx = jnp.clip(_round_i32(sx / safe), 0, _W - 1)
        y_o = ycs[c]
        x_o = xcs[c]
        r_o = ccr[c]
        g_o = ccg[c]
        b_o = ccb[c]
        y_n = jnp.where(nz, ny, y_o)
        x_n = jnp.where(nz, nx, x_o)
        r_n = jnp.where(nz, sr / safe, r_o)
        g_n = jnp.where(nz, sg / safe, g_o)
        b_n = jnp.where(nz, sb / safe, b_o)
        ycs[c] = y_n
        xcs[c] = x_n
        ccr[c] = r_n
        ccg[c] = g_n
        ccb[c] = b_n
        dyf = (y_n - y_o).astype(jnp.float32)
        dxf = (x_n - x_o).astype(jnp.float32)
        drf = r_n - r_o
        dgf = g_n - g_o
        dbf = b_n - b_o
        d2 = ms * (dyf * dyf + dxf * dxf) + (drf * drf + dgf * dgf
                                             + dbf * dbf)
        dm_ref[0] = jnp.maximum(dm_ref[0], d2)

    _WROW = {
        w: jax.lax.broadcasted_iota(jnp.int32, (w, _W), 0) for w in (40, 64)
    }
    _WCOLF = {
        w: jax.lax.broadcasted_iota(jnp.int32, (w, _W), 1).astype(
            jnp.float32) for w in (40, 64)
    }

    def seg_pass():
        bmax = jnp.max(best_ref[:, :])
        small40 = bmax <= jnp.float32(_MS * 16 * 16)
        small64 = bmax <= jnp.float32(_MS * 28 * 28)
        dm_ref[0] = jnp.float32(0.0)

        def _win_sums(c, win, rad):
            y = ycs[c]
            st = jnp.clip(((y - rad) // 8) * 8, 0, _H - win)
            st = pl.multiple_of(st, 8)
            m = lab_ref[pl.ds(st, win), :] == c
            rowfw = (_WROW[win] + st).astype(jnp.float32)
            cnt = jnp.sum(jnp.where(m, 1.0, 0.0))
            sy = jnp.sum(jnp.where(m, rowfw, 0.0))
            sx = jnp.sum(jnp.where(m, _WCOLF[win], 0.0))
            sr = jnp.sum(jnp.where(m, x_ref[0, pl.ds(st, win), :], 0.0))
            sg = jnp.sum(jnp.where(m, x_ref[1, pl.ds(st, win), :], 0.0))
            sb = jnp.sum(jnp.where(m, x_ref[2, pl.ds(st, win), :], 0.0))
            return cnt, sy, sx, sr, sg, sb

        def c_body_w40(k, _):
            base = k * 7
            sums = [_win_sums(base + t, 40, 16) for t in range(7)]
            for t in range(7):
                _seg_update(base + t, *sums[t])
            return 0

        def c_body_win(k, _):
            base = k * 4
            sums = [_win_sums(base + t, 64, 28) for t in range(4)]
            for t in range(4):
                _seg_update(base + t, *sums[t])
            return 0

        def c_body_full(c, _):
            m = lab_ref[:, :] == c
            cnt = jnp.sum(jnp.where(m, 1.0, 0.0))
            sy = jnp.sum(jnp.where(m, rowf, 0.0))
            sx = jnp.sum(jnp.where(m, colf, 0.0))
            sr = jnp.sum(jnp.where(m, xr, 0.0))
            sg = jnp.sum(jnp.where(m, xg, 0.0))
            sb = jnp.sum(jnp.where(m, xb_, 0.0))
            _seg_update(c, cnt, sy, sx, sr, sg, sb)
            return 0

        jax.lax.cond(
            small40,
            lambda: jax.lax.fori_loop(0, _C // 7, c_body_w40, 0),
            lambda: jax.lax.cond(
                small64,
                lambda: jax.lax.fori_loop(0, _C // 4, c_body_win, 0),
                lambda: jax.lax.fori_loop(0, _C, c_body_full, 0),
            ),
        )
        sroot = _sqrt_s(bmax) + _sqrt_s(dm_ref[0])
        u_ref[0] = sroot * sroot * jnp.float32(1.01) + jnp.float32(1.0)

    u_ref[0] = jnp.float32(260.0)
    dm_ref[0] = jnp.float32(1.0)

    def it_body(t, _):
        jax.lax.cond(
            dm_ref[0] != 0.0,
            lambda: (dist_pass(), seg_pass(), 0)[2],
            lambda: 0,
        )
        return 0

    jax.lax.fori_loop(0, _ITERS - 1, it_body, 0)
    dist_pass()
    out_ref[0] = lab_ref[:, :]


def kernel(x, grad_map):
    if grad_map.ndim == 3:
        grad_map = grad_map[:, None]
    f = pl.pallas_call(
        _slic_kernel,
        out_shape=jax.ShapeDtypeStruct((1, _H, _W), jnp.int32),
        scratch_shapes=[
            pltpu.VMEM((_H, _W), jnp.int32),
            pltpu.VMEM((_H, _W), jnp.float32),
            pltpu.VMEM((_H, _W), jnp.int32),
            pltpu.SMEM((_C,), jnp.int32),
            pltpu.SMEM((_C,), jnp.int32),
            pltpu.SMEM((_C,), jnp.float32),
            pltpu.SMEM((_C,), jnp.float32),
            pltpu.SMEM((_C,), jnp.float32),
            pltpu.SMEM((1,), jnp.float32),
            pltpu.SMEM((1,), jnp.float32),
            pltpu.SMEM((8,), jnp.int32),
            pltpu.SMEM((_H // 32, _C), jnp.int32),
        ],
    )
    return f(x[0], grad_map[0, 0])
